# Initial kernel scaffold; baseline (speedup 1.0000x reference)
import jax
import jax.numpy as jnp
from jax.experimental import pallas as pl


def kernel(x, edge_index, W1, b1, W2, b2, Wf, bf):
    raise NotImplementedError("write your pallas kernel here")


# speedup vs baseline: 14.2378x; 14.2378x over previous
import functools

import jax
import jax.numpy as jnp
from jax import lax
from jax.experimental import pallas as pl
from jax.experimental.pallas import tpu as pltpu
from jax.experimental.pallas import tpu_sc as plsc

N = 10000
E = 320000
D = 128

NC = 2
NS = 16
NW = NC * NS

CHUNK = 128
K = 79
E_PAD = NW * K * CHUNK
N_ACC = 10112
RPT = N_ACC // NS

NBLK = 8
BLK = N_ACC // NBLK


_MESH = plsc.VectorSubcoreMesh(
    core_axis_name="c", subcore_axis_name="s", num_cores=NC, num_subcores=NS
)


def _sc_hist_body(dst_hbm, ones_hbm, zeros_hbm, out_hbm, dst_v, ones_v, acc, sem):
    c = lax.axis_index("c")
    s = lax.axis_index("s")
    w = c * NS + s
    pltpu.sync_copy(dst_hbm.at[w], dst_v)
    pltpu.sync_copy(ones_hbm, ones_v)

    @pl.when(s == 0)
    def _():
        pltpu.sync_copy(zeros_hbm, acc)

    plsc.subcore_barrier()

    def chunk(j):
        pltpu.sync_copy(ones_v, acc.at[dst_v.at[j]], add=True)

    pl.loop(0, K)(chunk)
    plsc.subcore_barrier()

    @pl.when(s == 0)
    def _():
        pltpu.sync_copy(acc, out_hbm.at[c])


_sc_hist = pl.kernel(
    _sc_hist_body,
    out_type=jax.ShapeDtypeStruct((NC, N_ACC), jnp.float32),
    mesh=_MESH,
    scratch_types=[
        pltpu.VMEM((K, CHUNK), jnp.int32),
        pltpu.VMEM((CHUNK,), jnp.float32),
        pltpu.VMEM_SHARED((N_ACC,), jnp.float32),
        pltpu.SemaphoreType.DMA,
    ],
)


def _sc_agg_body(h_hbm, src_hbm, dst_hbm, zeros_hbm, out_hbm,
                 src_v, dst_v, rows_v, acc, sem):
    c = lax.axis_index("c")
    s = lax.axis_index("s")
    w = c * NS + s
    pltpu.sync_copy(zeros_hbm, acc.at[pl.ds(s * RPT, RPT)])
    pltpu.sync_copy(src_hbm.at[w], src_v)
    pltpu.sync_copy(dst_hbm.at[w], dst_v)
    plsc.subcore_barrier()

    def chunk(j):
        pltpu.async_copy(h_hbm.at[src_v.at[j]], rows_v, sem).wait()
        pltpu.sync_copy(rows_v, acc.at[dst_v.at[j]], add=True)

    pl.loop(0, K)(chunk)
    plsc.subcore_barrier()
    pltpu.sync_copy(acc.at[pl.ds(s * RPT, RPT)], out_hbm.at[c, pl.ds(s * RPT, RPT)])


_sc_agg = pl.kernel(
    _sc_agg_body,
    out_type=jax.ShapeDtypeStruct((NC, N_ACC, D), jnp.float32),
    mesh=_MESH,
    scratch_types=[
        pltpu.VMEM((K, CHUNK), jnp.int32),
        pltpu.VMEM((K, CHUNK), jnp.int32),
        pltpu.VMEM((CHUNK, D), jnp.float32),
        pltpu.VMEM_SHARED((N_ACC, D), jnp.float32),
        pltpu.SemaphoreType.DMA,
    ],
)


def _dis(p_ref):
    deg = p_ref[:, 0:1] + p_ref[:, 1:2] + 1.0
    return lax.rsqrt(deg)


def _tc1_body(p_ref, x_ref, w_ref, o_ref):
    o_ref[...] = _dis(p_ref) * jnp.dot(
        x_ref[...], w_ref[...], preferred_element_type=jnp.float32
    )


def _tc2_body(a_ref, h_ref, p_ref, b_ref, w_ref, o_ref):
    dis = _dis(p_ref)
    z = jnp.maximum(dis * (a_ref[0] + a_ref[1] + h_ref[...]) + b_ref[...], 0.0)
    o_ref[...] = dis * jnp.dot(z, w_ref[...], preferred_element_type=jnp.float32)


def _tc3_body(a_ref, h_ref, p_ref, b_ref, wf_ref, bf_ref, o_ref):
    dis = _dis(p_ref)
    z = jnp.maximum(dis * (a_ref[0] + a_ref[1] + h_ref[...]) + b_ref[...], 0.0)
    o_ref[...] = jnp.tanh(
        jnp.dot(z, wf_ref[...], preferred_element_type=jnp.float32) + bf_ref[...]
    )


_spec_p = pl.BlockSpec((BLK, NC), lambda i: (i, 0))
_spec_a = pl.BlockSpec((NC, BLK, D), lambda i: (0, i, 0))
_spec_n = pl.BlockSpec((BLK, D), lambda i: (i, 0))
_spec_w = pl.BlockSpec((D, D), lambda i: (0, 0))
_spec_b = pl.BlockSpec((1, D), lambda i: (0, 0))
_out_n = jax.ShapeDtypeStruct((N_ACC, D), jnp.float32)

_tc1 = pl.pallas_call(
    _tc1_body, grid=(NBLK,),
    in_specs=[_spec_p, _spec_n, _spec_w],
    out_specs=_spec_n, out_shape=_out_n,
)
_tc2 = pl.pallas_call(
    _tc2_body, grid=(NBLK,),
    in_specs=[_spec_a, _spec_n, _spec_p, _spec_b, _spec_w],
    out_specs=_spec_n, out_shape=_out_n,
)
_tc3 = pl.pallas_call(
    _tc3_body, grid=(NBLK,),
    in_specs=[_spec_a, _spec_n, _spec_p, _spec_b, _spec_w, _spec_b],
    out_specs=_spec_n, out_shape=_out_n,
)


def kernel(x, edge_index, W1, b1, W2, b2, Wf, bf):
    src = edge_index[0].astype(jnp.int32)
    dst = edge_index[1].astype(jnp.int32)

    pad = E_PAD - E
    src_p = jnp.concatenate([src, jnp.full((pad,), N, jnp.int32)])
    dst_p = jnp.concatenate(
        [dst, N + (jnp.arange(pad, dtype=jnp.int32) % (N_ACC - N))]
    )
    src_p = src_p.reshape(K, NW, CHUNK).transpose(1, 0, 2)
    dst_p = dst_p.reshape(K, NW, CHUNK).transpose(1, 0, 2)

    ones_c = jnp.ones((CHUNK,), jnp.float32)
    zeros_1 = jnp.zeros((N_ACC,), jnp.float32)
    zeros_d = jnp.zeros((RPT, D), jnp.float32)

    x_pad = jnp.pad(x, ((0, N_ACC - N), (0, 0)))
    b1r = b1.reshape(1, D)
    b2r = b2.reshape(1, D)
    wf_pad = jnp.zeros((D, D), jnp.float32).at[:, : Wf.shape[1]].set(Wf)
    bf_pad = jnp.zeros((1, D), jnp.float32).at[0, : bf.shape[0]].set(bf)

    p = _sc_hist(dst_p, ones_c, zeros_1).T
    h1 = _tc1(p, x_pad, W1)
    a1 = _sc_agg(h1, src_p, dst_p, zeros_d)
    h2 = _tc2(a1, h1, p, b1r, W2)
    a2 = _sc_agg(h2, src_p, dst_p, zeros_d)
    out = _tc3(a2, h2, p, b2r, wf_pad, bf_pad)
    return out[:N, : Wf.shape[1]]
